# baseline (speedup 1.0000x reference)
import functools

import jax
import jax.numpy as jnp
from jax import lax
from jax.experimental import pallas as pl
from jax.experimental.pallas import tpu as pltpu
from jax.experimental.pallas import tpu_sc as plsc


def _sc_gather(t, tab_a, tab_s):
    n = t.shape[0]
    info = plsc.get_sparse_core_info()
    nc, ns, nl = info.num_cores, info.num_subcores, info.num_lanes
    nw = nc * ns
    bpw = n // nw
    steps = tab_a.shape[0]
    mesh = plsc.VectorSubcoreMesh(core_axis_name="c", subcore_axis_name="s")

    @functools.partial(
        pl.kernel,
        mesh=mesh,
        compiler_params=pltpu.CompilerParams(needs_layout_passes=False),
        out_type=[
            jax.ShapeDtypeStruct((n,), jnp.float32),
            jax.ShapeDtypeStruct((n,), jnp.float32),
        ],
        scratch_types=[
            pltpu.VMEM((bpw,), jnp.int32),
            pltpu.VMEM((steps,), jnp.float32),
            pltpu.VMEM((steps,), jnp.float32),
            pltpu.VMEM((bpw,), jnp.float32),
            pltpu.VMEM((bpw,), jnp.float32),
            pltpu.SemaphoreType.DMA,
        ],
    )
    def gather_kernel(t_hbm, ta_hbm, ts_hbm, a_hbm, s_hbm,
                      idx_v, ta_v, ts_v, a_v, s_v, sem):
        wid = lax.axis_index("s") * nc + lax.axis_index("c")
        base = wid * bpw
        cp_idx = pltpu.async_copy(t_hbm.at[pl.ds(base, bpw)], idx_v, sem)
        cp_ta = pltpu.async_copy(ta_hbm, ta_v, sem)
        cp_ts = pltpu.async_copy(ts_hbm, ts_v, sem)
        cp_idx.wait()
        cp_ta.wait()
        cp_ts.wait()
        for i in range(bpw // nl):
            sl = pl.ds(i * nl, nl)
            idx = idx_v[sl]
            a_v[sl] = plsc.load_gather(ta_v, [idx])
            s_v[sl] = plsc.load_gather(ts_v, [idx])
        cp_a = pltpu.async_copy(a_v, a_hbm.at[pl.ds(base, bpw)], sem)
        cp_s = pltpu.async_copy(s_v, s_hbm.at[pl.ds(base, bpw)], sem)
        cp_a.wait()
        cp_s.wait()

    return gather_kernel(t, tab_a, tab_s)


def _tc_axpy(x0, noise, a, s):
    b, d = x0.shape
    br = 512
    nbuf = 4

    def outer(x0_hbm, n_hbm, a_ref, s_ref, o_hbm):
        def inner(x0_blk, n_blk, o_blk):
            i = pl.program_id(0)
            sl = pl.ds(i * br, br)
            av = a_ref[sl].reshape(br, 1)
            sv = s_ref[sl].reshape(br, 1)
            o_blk[...] = av * x0_blk[...] + sv * n_blk[...]

        pltpu.emit_pipeline(
            inner,
            grid=(b // br,),
            in_specs=[
                pl.BlockSpec((br, d), lambda i: (i, 0),
                             pipeline_mode=pl.Buffered(buffer_count=nbuf)),
                pl.BlockSpec((br, d), lambda i: (i, 0),
                             pipeline_mode=pl.Buffered(buffer_count=nbuf)),
            ],
            out_specs=[pl.BlockSpec((br, d), lambda i: (i, 0))],
        )(x0_hbm, n_hbm, o_hbm)

    return pl.pallas_call(
        outer,
        in_specs=[
            pl.BlockSpec(memory_space=pl.ANY),
            pl.BlockSpec(memory_space=pl.ANY),
            pl.BlockSpec(memory_space=pltpu.VMEM),
            pl.BlockSpec(memory_space=pltpu.VMEM),
        ],
        out_specs=pl.BlockSpec(memory_space=pl.ANY),
        out_shape=jax.ShapeDtypeStruct((b, d), jnp.float32),
    )(x0, noise, a, s)


def kernel(x0, noise, t, sqrt_alphas_cumprod, sqrt_1m_alphas_cumprod):
    a, s = _sc_gather(t, sqrt_alphas_cumprod, sqrt_1m_alphas_cumprod)
    return _tc_axpy(x0, noise, a, s)


# speedup vs baseline: 1.0058x; 1.0058x over previous
import functools

import jax
import jax.numpy as jnp
from jax import lax
from jax.experimental import pallas as pl
from jax.experimental.pallas import tpu as pltpu
from jax.experimental.pallas import tpu_sc as plsc


def _sc_gather(t, tab_a, tab_s):
    n = t.shape[0]
    info = plsc.get_sparse_core_info()
    nc, ns, nl = info.num_cores, info.num_subcores, info.num_lanes
    nw = nc * ns
    bpw = n // nw
    steps = tab_a.shape[0]
    mesh = plsc.VectorSubcoreMesh(core_axis_name="c", subcore_axis_name="s")

    @functools.partial(
        pl.kernel,
        mesh=mesh,
        compiler_params=pltpu.CompilerParams(needs_layout_passes=False),
        out_type=[
            jax.ShapeDtypeStruct((n,), jnp.float32),
            jax.ShapeDtypeStruct((n,), jnp.float32),
        ],
        scratch_types=[
            pltpu.VMEM((bpw,), jnp.int32),
            pltpu.VMEM((steps,), jnp.float32),
            pltpu.VMEM((steps,), jnp.float32),
            pltpu.VMEM((bpw,), jnp.float32),
            pltpu.VMEM((bpw,), jnp.float32),
            pltpu.SemaphoreType.DMA,
        ],
    )
    def gather_kernel(t_hbm, ta_hbm, ts_hbm, a_hbm, s_hbm,
                      idx_v, ta_v, ts_v, a_v, s_v, sem):
        wid = lax.axis_index("s") * nc + lax.axis_index("c")
        base = wid * bpw
        cp_idx = pltpu.async_copy(t_hbm.at[pl.ds(base, bpw)], idx_v, sem)
        cp_ta = pltpu.async_copy(ta_hbm, ta_v, sem)
        cp_ts = pltpu.async_copy(ts_hbm, ts_v, sem)
        cp_idx.wait()
        cp_ta.wait()
        cp_ts.wait()
        for i in range(bpw // nl):
            sl = pl.ds(i * nl, nl)
            idx = idx_v[sl]
            a_v[sl] = plsc.load_gather(ta_v, [idx])
            s_v[sl] = plsc.load_gather(ts_v, [idx])
        cp_a = pltpu.async_copy(a_v, a_hbm.at[pl.ds(base, bpw)], sem)
        cp_s = pltpu.async_copy(s_v, s_hbm.at[pl.ds(base, bpw)], sem)
        cp_a.wait()
        cp_s.wait()

    return gather_kernel(t, tab_a, tab_s)


def _tc_axpy(x0, noise, a, s):
    b, d = x0.shape
    br = 512

    def body(x0_ref, n_ref, a_ref, s_ref, o_ref):
        i = pl.program_id(0)
        sl = pl.ds(i * br, br)
        av = a_ref[sl].reshape(br, 1)
        sv = s_ref[sl].reshape(br, 1)
        o_ref[...] = av * x0_ref[...] + sv * n_ref[...]

    return pl.pallas_call(
        body,
        grid=(b // br,),
        in_specs=[
            pl.BlockSpec((br, d), lambda i: (i, 0)),
            pl.BlockSpec((br, d), lambda i: (i, 0)),
            pl.BlockSpec(memory_space=pltpu.VMEM),
            pl.BlockSpec(memory_space=pltpu.VMEM),
        ],
        out_specs=pl.BlockSpec((br, d), lambda i: (i, 0)),
        out_shape=jax.ShapeDtypeStruct((b, d), jnp.float32),
    )(x0, noise, a, s)


def kernel(x0, noise, t, sqrt_alphas_cumprod, sqrt_1m_alphas_cumprod):
    a, s = _sc_gather(t, sqrt_alphas_cumprod, sqrt_1m_alphas_cumprod)
    return _tc_axpy(x0, noise, a, s)
